# baseline (speedup 1.0000x reference)
import functools

import jax
import jax.numpy as jnp
from jax import lax
from jax.experimental import pallas as pl
from jax.experimental.pallas import tpu as pltpu
from jax.experimental.pallas import tpu_sc as plsc

NUM_HEADS = 32
NPOS = 64 * 64
LANES = 16
NVEC = NPOS // LANES


def _sc_gather(table, rank_flat, file_flat):
    mesh = plsc.VectorSubcoreMesh(core_axis_name="c", subcore_axis_name="s")

    @functools.partial(
        pl.kernel,
        mesh=mesh,
        out_type=jax.ShapeDtypeStruct((NUM_HEADS, 64, 64), jnp.float32),
        scratch_types=[
            pltpu.VMEM((15, 15), jnp.float32),
            pltpu.VMEM((NPOS,), jnp.int32),
            pltpu.VMEM((NPOS,), jnp.int32),
            pltpu.VMEM((64, 64), jnp.float32),
            pltpu.SemaphoreType.DMA,
            pltpu.SemaphoreType.DMA,
            pltpu.SemaphoreType.DMA,
        ],
        compiler_params=pltpu.CompilerParams(needs_layout_passes=False),
    )
    def run(table_hbm, rank_hbm, file_hbm, out_hbm,
            table_v, rank_v, file_v, out_v, sem_t, sem_r, sem_f):
        wid = lax.axis_index("s") * 2 + lax.axis_index("c")
        ct = pltpu.async_copy(table_hbm.at[wid], table_v, sem_t)
        cr = pltpu.async_copy(rank_hbm, rank_v, sem_r)
        cf = pltpu.async_copy(file_hbm, file_v, sem_f)
        ct.wait()
        cr.wait()
        cf.wait()

        for n in range(NVEC):
            rv = rank_v[pl.ds(n * LANES, LANES)]
            fv = file_v[pl.ds(n * LANES, LANES)]
            out_v[n // 4, pl.ds((n % 4) * LANES, LANES)] = (
                plsc.load_gather(table_v, [rv, fv]))

        pltpu.sync_copy(out_v, out_hbm.at[wid])

    return run(table, rank_flat, file_flat)


def kernel(bias_table, rank_idx, file_idx):
    return _sc_gather(bias_table, rank_idx.reshape(NPOS), file_idx.reshape(NPOS))


# speedup vs baseline: 1.0868x; 1.0868x over previous
import functools

import jax
import jax.numpy as jnp
from jax import lax
from jax.experimental import pallas as pl
from jax.experimental.pallas import tpu as pltpu
from jax.experimental.pallas import tpu_sc as plsc

NUM_HEADS = 32
NPOS = 64 * 64
LANES = 16
NGROUPS = NPOS // 64


def _sc_gather(table, packed):
    mesh = plsc.VectorSubcoreMesh(core_axis_name="c", subcore_axis_name="s")

    @functools.partial(
        pl.kernel,
        mesh=mesh,
        out_type=jax.ShapeDtypeStruct((NUM_HEADS, 64, 64), jnp.float32),
        scratch_types=[
            pltpu.VMEM((15, 15), jnp.float32),
            pltpu.VMEM((NPOS // 4,), jnp.int32),
            pltpu.VMEM((64, 64), jnp.float32),
            pltpu.SemaphoreType.DMA,
            pltpu.SemaphoreType.DMA,
            pltpu.SemaphoreType.DMA,
        ],
        compiler_params=pltpu.CompilerParams(needs_layout_passes=False),
    )
    def run(table_hbm, packed_hbm, out_hbm, table_v, pk_v, out_v,
            sem_t, sem_p, sem_o):
        wid = lax.axis_index("s") * 2 + lax.axis_index("c")
        ct = pltpu.async_copy(table_hbm.at[wid], table_v, sem_t)
        cp = pltpu.async_copy(packed_hbm, pk_v, sem_p)
        ct.wait()
        cp.wait()

        out_copies = []
        for g in range(NGROUPS):
            w = pk_v[pl.ds(g * LANES, LANES)]
            for j in range(4):
                b = lax.shift_right_logical(w, 8 * j) & 0xFF
                rb = b & 0xF
                fb = lax.shift_right_logical(b, 4)
                out_v[g, pl.ds(j * LANES, LANES)] = (
                    plsc.load_gather(table_v, [rb, fb]))
            if (g + 1) % (NGROUPS // 4) == 0:
                c = (g + 1) // (NGROUPS // 4) - 1
                rows = pl.ds(c * (NGROUPS // 4), NGROUPS // 4)
                out_copies.append(pltpu.async_copy(
                    out_v.at[rows], out_hbm.at[wid].at[rows], sem_o))
        for cpy in out_copies:
            cpy.wait()

    return run(table, packed)


def _pack_nibbles(rank_idx, file_idx):
    m = (rank_idx | (file_idx << 4)).reshape(NPOS // LANES, LANES)
    s = [m[j::4].reshape(NPOS // 4) for j in range(4)]
    return s[0] | (s[1] << 8) | (s[2] << 16) | (s[3] << 24)


def kernel(bias_table, rank_idx, file_idx):
    return _sc_gather(bias_table, _pack_nibbles(rank_idx, file_idx))


# speedup vs baseline: 1.0873x; 1.0004x over previous
import functools

import jax
import jax.numpy as jnp
from jax import lax
from jax.experimental import pallas as pl
from jax.experimental.pallas import tpu as pltpu
from jax.experimental.pallas import tpu_sc as plsc

NUM_HEADS = 32
NPOS = 64 * 64
LANES = 16
NGROUPS = NPOS // 64


def _sc_gather(table, packed):
    mesh = plsc.VectorSubcoreMesh(core_axis_name="c", subcore_axis_name="s")

    @functools.partial(
        pl.kernel,
        mesh=mesh,
        out_type=jax.ShapeDtypeStruct((NUM_HEADS, 64, 64), jnp.float32),
        scratch_types=[
            pltpu.VMEM((15, 15), jnp.float32),
            pltpu.VMEM((NPOS // 4,), jnp.int32),
            pltpu.VMEM((64, 64), jnp.float32),
            pltpu.SemaphoreType.DMA,
            pltpu.SemaphoreType.DMA,
            pltpu.SemaphoreType.DMA,
        ],
        compiler_params=pltpu.CompilerParams(needs_layout_passes=False),
    )
    def run(table_hbm, packed_hbm, out_hbm, table_v, pk_v, out_v,
            sem_t, sem_p, sem_o):
        wid = lax.axis_index("s") * 2 + lax.axis_index("c")
        ct = pltpu.async_copy(table_hbm.at[wid], table_v, sem_t)
        cp = pltpu.async_copy(packed_hbm, pk_v, sem_p)
        ct.wait()
        cp.wait()

        out_copies = []
        for g in range(NGROUPS):
            w = pk_v[pl.ds(g * LANES, LANES)]
            for j in range(4):
                b = lax.shift_right_logical(w, 8 * j) & 0xFF
                rb = b & 0xF
                fb = lax.shift_right_logical(b, 4)
                out_v[g, pl.ds(j * LANES, LANES)] = (
                    plsc.load_gather(table_v, [rb, fb]))
            if (g + 1) % (NGROUPS // 4) == 0:
                c = (g + 1) // (NGROUPS // 4) - 1
                rows = pl.ds(c * (NGROUPS // 4), NGROUPS // 4)
                out_copies.append(pltpu.async_copy(
                    out_v.at[rows], out_hbm.at[wid].at[rows], sem_o))
        for cpy in out_copies:
            cpy.wait()

    return run(table, packed)


def _pack_nibbles(rank_idx, file_idx):
    m = (rank_idx | (file_idx << 4)).reshape(NGROUPS, 4, LANES)
    w = m[:, 0] | (m[:, 1] << 8) | (m[:, 2] << 16) | (m[:, 3] << 24)
    return w.reshape(NPOS // 4)


def kernel(bias_table, rank_idx, file_idx):
    return _sc_gather(bias_table, _pack_nibbles(rank_idx, file_idx))


# speedup vs baseline: 1.0906x; 1.0031x over previous
import functools

import jax
import jax.numpy as jnp
from jax import lax
from jax.experimental import pallas as pl
from jax.experimental.pallas import tpu as pltpu
from jax.experimental.pallas import tpu_sc as plsc

NUM_HEADS = 32
NPOS = 64 * 64
LANES = 16
NGROUPS = NPOS // 64


def _sc_gather(table, packed):
    mesh = plsc.VectorSubcoreMesh(core_axis_name="c", subcore_axis_name="s")

    @functools.partial(
        pl.kernel,
        mesh=mesh,
        out_type=jax.ShapeDtypeStruct((NUM_HEADS, 64, 64), jnp.float32),
        scratch_types=[
            pltpu.VMEM((15, 15), jnp.float32),
            pltpu.VMEM((NPOS // 4,), jnp.int32),
            pltpu.VMEM((64, 64), jnp.float32),
            pltpu.SemaphoreType.DMA,
            pltpu.SemaphoreType.DMA,
        ],
        compiler_params=pltpu.CompilerParams(needs_layout_passes=False),
    )
    def run(table_hbm, packed_hbm, out_hbm, table_v, pk_v, out_v, sem_t, sem_p):
        wid = lax.axis_index("s") * 2 + lax.axis_index("c")
        ct = pltpu.async_copy(table_hbm.at[wid], table_v, sem_t)
        cp = pltpu.async_copy(packed_hbm, pk_v, sem_p)
        ct.wait()
        cp.wait()

        for g in range(NGROUPS):
            w = pk_v[pl.ds(g * LANES, LANES)]
            for j in range(4):
                b = lax.shift_right_logical(w, 8 * j) & 0xFF
                rb = b & 0xF
                fb = lax.shift_right_logical(b, 4)
                out_v[g, pl.ds(j * LANES, LANES)] = (
                    plsc.load_gather(table_v, [rb, fb]))

        pltpu.sync_copy(out_v, out_hbm.at[wid])

    return run(table, packed)


def _pack_nibbles(rank_idx, file_idx):
    m = (rank_idx | (file_idx << 4)).reshape(NGROUPS, 4, LANES)
    w = m[:, 0] | (m[:, 1] << 8) | (m[:, 2] << 16) | (m[:, 3] << 24)
    return w.reshape(NPOS // 4)


def kernel(bias_table, rank_idx, file_idx):
    return _sc_gather(bias_table, _pack_nibbles(rank_idx, file_idx))


# speedup vs baseline: 1.0928x; 1.0021x over previous
import functools

import jax
import jax.numpy as jnp
from jax import lax
from jax.experimental import pallas as pl
from jax.experimental.pallas import tpu as pltpu
from jax.experimental.pallas import tpu_sc as plsc

NUM_HEADS = 32
NPOS = 64 * 64
LANES = 16
NGROUPS = NPOS // 64


def _sc_gather(table, packed):
    mesh = plsc.VectorSubcoreMesh(core_axis_name="c", subcore_axis_name="s")

    @functools.partial(
        pl.kernel,
        mesh=mesh,
        out_type=jax.ShapeDtypeStruct((NUM_HEADS, 64, 64), jnp.float32),
        scratch_types=[
            pltpu.VMEM((15, 15), jnp.float32),
            pltpu.VMEM((8, 128), jnp.int32),
            pltpu.VMEM((64, 64), jnp.float32),
            pltpu.SemaphoreType.DMA,
            pltpu.SemaphoreType.DMA,
        ],
        compiler_params=pltpu.CompilerParams(needs_layout_passes=False),
    )
    def run(table_hbm, packed_hbm, out_hbm, table_v, pk_v, out_v, sem_t, sem_p):
        wid = lax.axis_index("s") * 2 + lax.axis_index("c")
        ct = pltpu.async_copy(table_hbm.at[wid], table_v, sem_t)
        cp = pltpu.async_copy(packed_hbm, pk_v, sem_p)
        ct.wait()
        cp.wait()

        for g in range(NGROUPS):
            w = pk_v[g // 8, pl.ds((g % 8) * LANES, LANES)]
            for j in range(4):
                b = lax.shift_right_logical(w, 8 * j) & 0xFF
                rb = b & 0xF
                fb = lax.shift_right_logical(b, 4)
                out_v[g, pl.ds(j * LANES, LANES)] = (
                    plsc.load_gather(table_v, [rb, fb]))

        pltpu.sync_copy(out_v, out_hbm.at[wid])

    return run(table, packed)


def _pack_nibbles(rank_idx, file_idx):
    m4 = (rank_idx | (file_idx << 4)).reshape(8, 8, 4, LANES)
    w = (m4[:, :, 0] | (m4[:, :, 1] << 8)
         | (m4[:, :, 2] << 16) | (m4[:, :, 3] << 24))
    return w.reshape(8, 128)


def kernel(bias_table, rank_idx, file_idx):
    return _sc_gather(bias_table, _pack_nibbles(rank_idx, file_idx))
